# Initial kernel scaffold; baseline (speedup 1.0000x reference)
import jax
import jax.numpy as jnp
from jax.experimental import pallas as pl


def kernel(adj):
    raise NotImplementedError("write your pallas kernel here")


# speedup vs baseline: 396.2673x; 396.2673x over previous
import functools

import jax
import jax.numpy as jnp
from jax import lax
from jax.experimental import pallas as pl
from jax.experimental.pallas import tpu as pltpu

_SPARSITY_RATIO = 0.3


def _fold_sum_i16(m):
    w = m.shape[1]
    while w > 128:
        w //= 2
        m = m[:, :w] + m[:, w:]
    return jnp.sum(m.astype(jnp.float32), axis=1, keepdims=True)


def _bisect_i16(v, tgt):
    def cnt_ge(tc32):
        tc = tc32.astype(jnp.int16)
        return _fold_sum_i16((v >= tc).astype(jnp.int16))

    r = v.shape[0]
    zero = jnp.zeros((r, 1), jnp.int32)
    t = jnp.where(cnt_ge(zero) >= tgt, zero, jnp.full_like(zero, -32768))
    for b in range(14, -1, -1):
        tc = t + jnp.int32(1 << b)
        t = jnp.where(cnt_ge(tc) >= tgt, tc, t)
    return t


def _select_mask_kernel(x_ref, o_ref, *, k):
    x = x_ref[0]
    u = lax.bitcast_convert_type(x, jnp.int32)
    key = u ^ ((u >> 31) & jnp.int32(0x7FFFFFFF))
    khi = (key >> 16).astype(jnp.int16)
    klo = ((key & jnp.int32(0xFFFF)) - jnp.int32(32768)).astype(jnp.int16)

    kf = jnp.float32(k)
    thi = _bisect_i16(khi, kf)
    thi16 = thi.astype(jnp.int16)
    klo2 = jnp.where(khi == thi16, klo,
                     jnp.where(khi > thi16, jnp.int16(32767),
                               jnp.int16(-32768)))
    tlo = _bisect_i16(klo2, kf)

    t32 = (thi << 16) | (tlo + jnp.int32(32768))
    o_ref[0] = jnp.where(key >= t32, x, jnp.float32(0.0))


@jax.jit
def kernel(adj):
    B, R, N = adj.shape
    k = max(1, int(N * (1.0 - _SPARSITY_RATIO)))
    RBLK = 256
    grid = (B, R // RBLK)
    return pl.pallas_call(
        functools.partial(_select_mask_kernel, k=k),
        grid=grid,
        in_specs=[pl.BlockSpec((1, RBLK, N), lambda b, r: (b, r, 0))],
        out_specs=pl.BlockSpec((1, RBLK, N), lambda b, r: (b, r, 0)),
        out_shape=jax.ShapeDtypeStruct(adj.shape, adj.dtype),
        compiler_params=pltpu.CompilerParams(
            dimension_semantics=("parallel", "parallel"),
        ),
    )(adj)


# speedup vs baseline: 396.5949x; 1.0008x over previous
import functools

import jax
import jax.numpy as jnp
from jax import lax
from jax.experimental import pallas as pl
from jax.experimental.pallas import tpu as pltpu

_SPARSITY_RATIO = 0.3


def _fold_sum_i16(m):
    w = m.shape[1]
    while w > 128:
        w //= 2
        m = m[:, :w] + m[:, w:]
    return jnp.sum(m.astype(jnp.float32), axis=1, keepdims=True)


def _bisect_i16(v, tgt):
    def cnt_ge(tc32):
        tc = tc32.astype(jnp.int16)
        return _fold_sum_i16((v >= tc).astype(jnp.int16))

    r = v.shape[0]
    zero = jnp.zeros((r, 1), jnp.int32)
    t = jnp.where(cnt_ge(zero) >= tgt, zero, jnp.full_like(zero, -32768))
    for b in range(14, -1, -1):
        tc = t + jnp.int32(1 << b)
        t = jnp.where(cnt_ge(tc) >= tgt, tc, t)
    return t


def _select_mask_kernel(x_ref, o_ref, *, k):
    x = x_ref[0]
    u = lax.bitcast_convert_type(x, jnp.int32)
    key = u ^ ((u >> 31) & jnp.int32(0x7FFFFFFF))
    khi = (key >> 16).astype(jnp.int16)
    klo = ((key & jnp.int32(0xFFFF)) - jnp.int32(32768)).astype(jnp.int16)

    kf = jnp.float32(k)
    thi = _bisect_i16(khi, kf)
    thi16 = thi.astype(jnp.int16)
    klo2 = jnp.where(khi == thi16, klo,
                     jnp.where(khi > thi16, jnp.int16(32767),
                               jnp.int16(-32768)))
    tlo = _bisect_i16(klo2, kf)

    t32 = (thi << 16) | (tlo + jnp.int32(32768))
    o_ref[0] = jnp.where(key >= t32, x, jnp.float32(0.0))


@jax.jit
def kernel(adj):
    B, R, N = adj.shape
    k = max(1, int(N * (1.0 - _SPARSITY_RATIO)))
    RBLK = 512
    grid = (B, R // RBLK)
    return pl.pallas_call(
        functools.partial(_select_mask_kernel, k=k),
        grid=grid,
        in_specs=[pl.BlockSpec((1, RBLK, N), lambda b, r: (b, r, 0))],
        out_specs=pl.BlockSpec((1, RBLK, N), lambda b, r: (b, r, 0)),
        out_shape=jax.ShapeDtypeStruct(adj.shape, adj.dtype),
        compiler_params=pltpu.CompilerParams(
            dimension_semantics=("parallel", "parallel"),
        ),
    )(adj)
